# baseline (speedup 1.0000x reference)
import functools

import jax
import jax.numpy as jnp
from jax import lax
from jax.experimental import pallas as pl
from jax.experimental.pallas import tpu as pltpu
from jax.experimental.pallas import tpu_sc as plsc

N_CODES = 8192
D = 32
B = 16
HW = 1024
NB = 8192
NCH = N_CODES // NB


def _argmax_body(z_ref, e_ref, ind_ref):
    zb = z_ref[0]
    eb = e_ref[...]
    logits = lax.dot_general(eb, zb, (((1,), (0,)), ((), ())),
                             preferred_element_type=jnp.float32)
    ind_ref[0] = jnp.argmax(logits, axis=0)[None, :].astype(jnp.int32)


HWC = 2


def _argmax_call(z3, embedding):
    nb = z3.shape[0]
    hb = HW // HWC
    return pl.pallas_call(
        _argmax_body,
        grid=(nb, HWC),
        in_specs=[
            pl.BlockSpec((1, D, hb), lambda b, h: (b, 0, h)),
            pl.BlockSpec((N_CODES, D), lambda b, h: (0, 0)),
        ],
        out_specs=pl.BlockSpec((1, 1, hb), lambda b, h: (b, 0, h)),
        out_shape=jax.ShapeDtypeStruct((nb, 1, HW), jnp.int32),
    )(z3, embedding)


_NW = 32


@functools.lru_cache(maxsize=None)
def _sc_gather_fn(n_idx):
    bpw = n_idx // _NW

    @functools.partial(
        pl.kernel,
        mesh=plsc.VectorSubcoreMesh(core_axis_name="c", subcore_axis_name="s"),
        out_type=jax.ShapeDtypeStruct((n_idx, D), jnp.float32),
        scratch_types=[
            pltpu.VMEM((bpw,), jnp.int32),
            pltpu.VMEM((bpw, D), jnp.float32),
            pltpu.SemaphoreType.DMA,
        ],
        compiler_params=pltpu.CompilerParams(use_tc_tiling_on_sc=False),
    )
    def _sc_gather(table_hbm, idx_hbm, out_hbm, idx_v, rows_v, sem):
        wid = lax.axis_index("s") * 2 + lax.axis_index("c")
        base = wid * bpw
        pltpu.sync_copy(idx_hbm.at[pl.ds(base, bpw)], idx_v)
        pltpu.async_copy(table_hbm.at[idx_v], rows_v, sem).wait()
        pltpu.sync_copy(rows_v, out_hbm.at[pl.ds(base, bpw)])

    return _sc_gather


def kernel(z, embedding):
    z3 = z.reshape(B, D, HW)
    ind = _argmax_call(z3, embedding).reshape(-1)
    rows = _sc_gather_fn(B * HW)(embedding, ind)
    z_q = rows.reshape(B, HW, D).transpose(0, 2, 1).reshape(B, D, 32, 32)
    return (z_q, ind)


# speedup vs baseline: 1.0303x; 1.0303x over previous
import functools

import jax
import jax.numpy as jnp
from jax import lax
from jax.experimental import pallas as pl
from jax.experimental.pallas import tpu as pltpu
from jax.experimental.pallas import tpu_sc as plsc

N_CODES = 8192
D = 32
B = 16
HW = 1024
NB = 8192
NCH = N_CODES // NB


def _argmax_body(z_ref, e_ref, ind_ref):
    zb = z_ref[0]
    eb = e_ref[...]
    logits = lax.dot_general(eb, zb, (((1,), (0,)), ((), ())),
                             preferred_element_type=jnp.float32)
    ind_ref[0] = jnp.argmax(logits, axis=0)[None, :].astype(jnp.int32)


def _argmax_call(z3, embedding):
    nb = z3.shape[0]
    return pl.pallas_call(
        _argmax_body,
        grid=(nb,),
        in_specs=[
            pl.BlockSpec((1, D, HW), lambda b: (b, 0, 0)),
            pl.BlockSpec((N_CODES, D), lambda b: (0, 0)),
        ],
        out_specs=pl.BlockSpec((1, 1, HW), lambda b: (b, 0, 0)),
        out_shape=jax.ShapeDtypeStruct((nb, 1, HW), jnp.int32),
    )(z3, embedding)


_NW = 32


@functools.lru_cache(maxsize=None)
def _sc_gather_fn(n_idx):
    bpw = n_idx // _NW

    @functools.partial(
        pl.kernel,
        mesh=plsc.VectorSubcoreMesh(core_axis_name="c", subcore_axis_name="s"),
        out_type=jax.ShapeDtypeStruct((n_idx, D), jnp.float32),
        scratch_types=[
            pltpu.VMEM((bpw,), jnp.int32),
            pltpu.VMEM((bpw, D), jnp.float32),
            pltpu.SemaphoreType.DMA,
        ],
        compiler_params=pltpu.CompilerParams(use_tc_tiling_on_sc=False),
    )
    def _sc_gather(table_hbm, idx_hbm, out_hbm, idx_v, rows_v, sem):
        wid = lax.axis_index("s") * 2 + lax.axis_index("c")
        base = wid * bpw
        pltpu.sync_copy(idx_hbm.at[pl.ds(base, bpw)], idx_v)
        pltpu.async_copy(table_hbm.at[idx_v], rows_v, sem).wait()
        pltpu.sync_copy(rows_v, out_hbm.at[pl.ds(base, bpw)])

    return _sc_gather


def kernel(z, embedding):
    z3 = z.reshape(B, D, HW)
    ind = _argmax_call(z3, embedding).reshape(-1)
    rows = _sc_gather_fn(B * HW)(embedding, ind)
    z_q = rows.reshape(B, HW, D).transpose(0, 2, 1).reshape(B, D, 32, 32)
    return (z_q, ind)
